# Initial kernel scaffold; baseline (speedup 1.0000x reference)
import jax
import jax.numpy as jnp
from jax.experimental import pallas as pl


def kernel(x):
    raise NotImplementedError("write your pallas kernel here")


# speedup vs baseline: 10.9847x; 10.9847x over previous
import jax
import jax.numpy as jnp
from jax import lax
from jax.experimental import pallas as pl
from jax.experimental.pallas import tpu as pltpu
from jax.experimental.pallas import tpu_sc as plsc

B = 2
T = 8
C = 32
TC = T * C
H = W = 224
N = H * W
K = N // 2
P = 512
NB = N // P
BN = B * N
BK = B * K
KP = K + P
SC_WIN = 128

def _vmesh():
    return plsc.VectorSubcoreMesh(core_axis_name="c", subcore_axis_name="s")


def _k1_body(x_ref, xf_ref, e_ref):
    xb = x_ref[0]
    xf_ref[0] = xb.T
    sq = xb * xb
    r3 = sq.reshape(T, C, P)
    s = jnp.sum(r3, axis=1)
    rt = jnp.sqrt(s)
    e_ref[0, 0] = jnp.sum(rt, axis=0, keepdims=True) * (1.0 / T)


def _energy_and_transpose(xv):
    return pl.pallas_call(
        _k1_body,
        grid=(B, NB),
        in_specs=[pl.BlockSpec((1, TC, P), lambda b, j: (b, 0, j))],
        out_specs=[
            pl.BlockSpec((1, P, TC), lambda b, j: (b, j, 0)),
            pl.BlockSpec((1, 1, 1, P), lambda b, j: (b, j, 0, 0)),
        ],
        out_shape=[
            jax.ShapeDtypeStruct((B, N, TC), jnp.float32),
            jax.ShapeDtypeStruct((B, NB, 1, P), jnp.float32),
        ],
    )(xv)


def _k2a_body(e_ref, thr_ref):
    bits = lax.bitcast_convert_type(e_ref[0, :, 0, :], jnp.int32)

    def step(_, carry):
        lo, hi = carry
        mid = lo + (hi - lo) // 2
        g = jnp.sum((bits > mid).astype(jnp.int32))
        lo2 = jnp.where(g < K, lo, mid + 1)
        hi2 = jnp.where(g < K, mid, hi)
        return lo2, hi2

    lo0 = jnp.int32(0)
    hi0 = jnp.int32(0x7F800000)
    thr, _ = lax.fori_loop(0, 31, step, (lo0, hi0))
    cnt_gt = jnp.sum((bits > thr).astype(jnp.int32))
    r = K - cnt_gt
    lane = lax.broadcasted_iota(jnp.int32, (1, 8), 1)
    out = jnp.where(lane == 0, thr,
                    jnp.where(lane == 1, cnt_gt,
                              jnp.where(lane == 2, r, 0)))
    thr_ref[0] = out


def _find_threshold(e4):
    return pl.pallas_call(
        _k2a_body,
        grid=(B,),
        in_specs=[pl.BlockSpec((1, NB, 1, P), lambda b: (b, 0, 0, 0))],
        out_specs=pl.BlockSpec((1, 1, 8), lambda b: (b, 0, 0)),
        out_shape=jax.ShapeDtypeStruct((B, 1, 8), jnp.int32),
    )(e4)


def _k2b_body(e_ref, thr_ref, comp_ref, carry_ref):
    j = pl.program_id(1)

    @pl.when(j == 0)
    def _():
        carry_ref[0] = 0
        carry_ref[1] = 0

    off = carry_ref[0]
    tcnt = carry_ref[1]
    thr = thr_ref[0, 0, 0]
    r = thr_ref[0, 0, 2]

    bits = lax.bitcast_convert_type(e_ref[0, 0], jnp.int32)
    m_gt = bits > thr
    m_eq = bits == thr

    io = lax.broadcasted_iota(jnp.int32, (P, P), 0)
    jo = lax.broadcasted_iota(jnp.int32, (P, P), 1)
    tri = jnp.where(io <= jo, 1.0, 0.0).astype(jnp.float32)

    peq = jnp.dot(m_eq.astype(jnp.float32), tri,
                  preferred_element_type=jnp.float32).astype(jnp.int32)
    take_tie = m_eq & ((tcnt + peq) <= r)
    mask = m_gt | take_tie
    sel = jnp.dot(mask.astype(jnp.float32), tri,
                  preferred_element_type=jnp.float32).astype(jnp.int32)

    sel_b = jnp.broadcast_to(sel - 1, (P, P))
    mask_b = jnp.broadcast_to(mask, (P, P))
    m_oh = jnp.where((sel_b == io) & mask_b, 1.0, 0.0).astype(jnp.float32)
    n_col = (j * P + lax.broadcasted_iota(jnp.int32, (P, 1), 0))
    comp = jnp.dot(m_oh, n_col.astype(jnp.float32),
                   precision=lax.Precision.HIGHEST,
                   preferred_element_type=jnp.float32).astype(jnp.int32)

    comp_ref[0, pl.ds(off, P)] = comp

    carry_ref[0] = off + jnp.sum(mask.astype(jnp.int32))
    carry_ref[1] = tcnt + jnp.sum(m_eq.astype(jnp.int32))


def _compact_indices(e4, thr):
    return pl.pallas_call(
        _k2b_body,
        grid=(B, NB),
        in_specs=[
            pl.BlockSpec((1, 1, 1, P), lambda b, j: (b, j, 0, 0)),
            pl.BlockSpec((1, 1, 8), lambda b, j: (b, 0, 0)),
        ],
        out_specs=pl.BlockSpec((1, KP, 1), lambda b, j: (b, 0, 0)),
        out_shape=jax.ShapeDtypeStruct((B, KP, 1), jnp.int32),
        scratch_shapes=[pltpu.SMEM((2,), jnp.int32)],
    )(e4, thr)


def _sc_gather_rows(x_flat2, gidx):
    @pl.kernel(
        out_type=jax.ShapeDtypeStruct((BK, TC), jnp.float32),
        mesh=_vmesh(),
    )
    def k(x_hbm, i_hbm, o_hbm):
        def body(i_vmem, o_vmem):
            pltpu.sync_copy(x_hbm.at[i_vmem.at[0]], o_vmem)

        pltpu.emit_pipeline(
            body,
            grid=(BK // SC_WIN,),
            in_specs=[pl.BlockSpec((1, SC_WIN), lambda i: (0, i))],
            out_specs=[pl.BlockSpec((SC_WIN, TC), lambda i: (i, 0))],
            core_axis_name=("c", "s"),
            dimension_semantics=(pltpu.PARALLEL,),
        )(i_hbm, o_hbm)

    return k(x_flat2, gidx)


def kernel(x):
    xv = x.reshape(B, TC, N)
    x_flat, e4 = _energy_and_transpose(xv)
    thr = _find_threshold(e4)
    comp = _compact_indices(e4, thr)

    top_indices = comp[:, :K, 0]
    gidx = (top_indices + jnp.arange(B, dtype=jnp.int32)[:, None] * N)
    gathered = _sc_gather_rows(x_flat.reshape(BN, TC), gidx.reshape(1, BK))
    x_sparse = gathered.reshape(B, K, T, C)
    return x_sparse, top_indices


# speedup vs baseline: 13.8929x; 1.2648x over previous
import jax
import jax.numpy as jnp
from jax import lax
from jax.experimental import pallas as pl
from jax.experimental.pallas import tpu as pltpu
from jax.experimental.pallas import tpu_sc as plsc

B = 2
T = 8
C = 32
TC = T * C
H = W = 224
N = H * W
K = N // 2
P = 512
NB = N // P
BN = B * N
BK = B * K
KP = K + P
SC_WIN = 128

def _vmesh():
    return plsc.VectorSubcoreMesh(core_axis_name="c", subcore_axis_name="s")


def _k1_body(x_ref, xf_ref, e_ref):
    xb = x_ref[0]
    xf_ref[0] = xb.T
    sq = xb * xb
    r3 = sq.reshape(T, C, P)
    s = jnp.sum(r3, axis=1)
    rt = jnp.sqrt(s)
    e_ref[0, 0] = jnp.sum(rt, axis=0, keepdims=True) * (1.0 / T)


def _energy_and_transpose(xv):
    return pl.pallas_call(
        _k1_body,
        grid=(B, NB),
        in_specs=[pl.BlockSpec((1, TC, P), lambda b, j: (b, 0, j))],
        out_specs=[
            pl.BlockSpec((1, P, TC), lambda b, j: (b, j, 0)),
            pl.BlockSpec((1, 1, 1, P), lambda b, j: (b, j, 0, 0)),
        ],
        out_shape=[
            jax.ShapeDtypeStruct((B, N, TC), jnp.float32),
            jax.ShapeDtypeStruct((B, NB, 1, P), jnp.float32),
        ],
    )(xv)


def _k2_body(e_ref, comp_ref, carry_ref, tri_ref, io_ref):
    j = pl.program_id(1)

    @pl.when(j == 0)
    def _():
        bits_all = lax.bitcast_convert_type(e_ref[0, :, 0, :], jnp.int32)

        def step(_, carry):
            lo, hi = carry
            mid = lo + (hi - lo) // 2
            g = jnp.sum((bits_all > mid).astype(jnp.int32))
            lo2 = jnp.where(g < K, lo, mid + 1)
            hi2 = jnp.where(g < K, mid, hi)
            return lo2, hi2

        thr, _ = lax.fori_loop(0, 31, step, (jnp.int32(0), jnp.int32(0x7F800000)))
        cnt_gt = jnp.sum((bits_all > thr).astype(jnp.int32))
        carry_ref[0] = 0
        carry_ref[1] = 0
        carry_ref[2] = thr
        carry_ref[3] = K - cnt_gt

        io = lax.broadcasted_iota(jnp.int32, (P, P), 0)
        jo = lax.broadcasted_iota(jnp.int32, (P, P), 1)
        io_ref[...] = io
        tri_ref[...] = jnp.where(io <= jo, 1.0, 0.0)

    off = carry_ref[0]
    tcnt = carry_ref[1]
    thr = carry_ref[2]
    r = carry_ref[3]

    bits = lax.bitcast_convert_type(e_ref[0, pl.ds(j, 1), 0, :], jnp.int32)
    m_gt = bits > thr
    m_eq = bits == thr

    both = jnp.concatenate(
        [m_gt.astype(jnp.float32), m_eq.astype(jnp.float32)], axis=0)
    pref = jnp.dot(both, tri_ref[...],
                   preferred_element_type=jnp.float32).astype(jnp.int32)
    pgt = pref[0:1]
    peq = pref[1:2]
    rr = jnp.maximum(r - tcnt, 0)
    sel = pgt + jnp.minimum(peq, rr)

    gmat = (jnp.broadcast_to(sel, (P, P)) > io_ref[...]).astype(jnp.float32)
    cnt = jnp.dot(gmat, jnp.ones((P, 1), jnp.float32),
                  preferred_element_type=jnp.float32).astype(jnp.int32)
    comp = (j * P + P) - cnt

    comp_ref[0, pl.ds(off, P)] = comp

    carry_ref[0] = off + sel[0, P - 1]
    carry_ref[1] = tcnt + peq[0, P - 1]


def _compact_indices(e4):
    return pl.pallas_call(
        _k2_body,
        grid=(B, NB),
        in_specs=[pl.BlockSpec((1, NB, 1, P), lambda b, j: (b, 0, 0, 0))],
        out_specs=pl.BlockSpec((1, KP, 1), lambda b, j: (b, 0, 0)),
        out_shape=jax.ShapeDtypeStruct((B, KP, 1), jnp.int32),
        scratch_shapes=[
            pltpu.SMEM((4,), jnp.int32),
            pltpu.VMEM((P, P), jnp.float32),
            pltpu.VMEM((P, P), jnp.int32),
        ],
    )(e4)


def _sc_gather_rows(x_flat2, gidx):
    @pl.kernel(
        out_type=jax.ShapeDtypeStruct((BK, TC), jnp.float32),
        mesh=_vmesh(),
    )
    def k(x_hbm, i_hbm, o_hbm):
        def body(i_vmem, o_vmem):
            pltpu.sync_copy(x_hbm.at[i_vmem.at[0]], o_vmem)

        pltpu.emit_pipeline(
            body,
            grid=(BK // SC_WIN,),
            in_specs=[pl.BlockSpec((1, SC_WIN), lambda i: (0, i))],
            out_specs=[pl.BlockSpec((SC_WIN, TC), lambda i: (i, 0))],
            core_axis_name=("c", "s"),
            dimension_semantics=(pltpu.PARALLEL,),
        )(i_hbm, o_hbm)

    return k(x_flat2, gidx)


def kernel(x):
    xv = x.reshape(B, TC, N)
    x_flat, e4 = _energy_and_transpose(xv)
    comp = _compact_indices(e4)

    top_indices = comp[:, :K, 0]
    gidx = (top_indices + jnp.arange(B, dtype=jnp.int32)[:, None] * N)
    gathered = _sc_gather_rows(x_flat.reshape(BN, TC), gidx.reshape(1, BK))
    x_sparse = gathered.reshape(B, K, T, C)
    return x_sparse, top_indices


# speedup vs baseline: 15.9136x; 1.1455x over previous
import jax
import jax.numpy as jnp
from jax import lax
from jax.experimental import pallas as pl
from jax.experimental.pallas import tpu as pltpu
from jax.experimental.pallas import tpu_sc as plsc

B = 2
T = 8
C = 32
TC = T * C
H = W = 224
N = H * W
K = N // 2
P = 512
NB = N // P
BN = B * N
BK = B * K
KP = K + P
SC_WIN = 128

def _vmesh():
    return plsc.VectorSubcoreMesh(core_axis_name="c", subcore_axis_name="s")


P1 = 3584
NB1 = N // P1
RP = P1 // P


def _k1_body(x_ref, xf_ref, e_ref):
    xb = x_ref[0]
    xf_ref[0] = xb.T
    sq = xb * xb
    r3 = sq.reshape(T, C, P1)
    s = jnp.sum(r3, axis=1)
    rt = jnp.sqrt(s)
    e = jnp.sum(rt, axis=0, keepdims=True) * (1.0 / T)
    for r in range(RP):
        e_ref[0, r] = e[:, r * P:(r + 1) * P]


def _energy_and_transpose(xv):
    return pl.pallas_call(
        _k1_body,
        grid=(B, NB1),
        in_specs=[pl.BlockSpec((1, TC, P1), lambda b, j: (b, 0, j))],
        out_specs=[
            pl.BlockSpec((1, P1, TC), lambda b, j: (b, j, 0)),
            pl.BlockSpec((1, RP, 1, P), lambda b, j: (b, j, 0, 0)),
        ],
        out_shape=[
            jax.ShapeDtypeStruct((B, N, TC), jnp.float32),
            jax.ShapeDtypeStruct((B, NB, 1, P), jnp.float32),
        ],
    )(xv)


def _k2_body(e_ref, comp_ref, carry_ref, tri_ref, io_ref):
    j = pl.program_id(1)

    @pl.when(j == 0)
    def _():
        bits_all = lax.bitcast_convert_type(e_ref[0, :, 0, :], jnp.int32)

        def step(_, carry):
            lo, hi = carry
            mid = lo + (hi - lo) // 2
            g = jnp.sum((bits_all > mid).astype(jnp.int32))
            lo2 = jnp.where(g < K, lo, mid + 1)
            hi2 = jnp.where(g < K, mid, hi)
            return lo2, hi2

        thr, _ = lax.fori_loop(0, 31, step, (jnp.int32(0), jnp.int32(0x7F800000)))
        cnt_gt = jnp.sum((bits_all > thr).astype(jnp.int32))
        carry_ref[0] = 0
        carry_ref[1] = 0
        carry_ref[2] = thr
        carry_ref[3] = K - cnt_gt

        io = lax.broadcasted_iota(jnp.int32, (P, P), 0)
        jo = lax.broadcasted_iota(jnp.int32, (P, P), 1)
        io_ref[...] = io
        tri_ref[...] = jnp.where(io <= jo, 1.0, 0.0)

    off = carry_ref[0]
    tcnt = carry_ref[1]
    thr = carry_ref[2]
    r = carry_ref[3]

    bits = lax.bitcast_convert_type(e_ref[0, pl.ds(j, 1), 0, :], jnp.int32)
    m_gt = bits > thr
    m_eq = bits == thr

    both = jnp.concatenate(
        [m_gt.astype(jnp.float32), m_eq.astype(jnp.float32)], axis=0)
    pref = jnp.dot(both, tri_ref[...],
                   preferred_element_type=jnp.float32).astype(jnp.int32)
    pgt = pref[0:1]
    peq = pref[1:2]
    rr = jnp.maximum(r - tcnt, 0)
    sel = pgt + jnp.minimum(peq, rr)

    gmat = (jnp.broadcast_to(sel, (P, P)) > io_ref[...]).astype(jnp.float32)
    cnt = jnp.dot(gmat, jnp.ones((P, 1), jnp.float32),
                  preferred_element_type=jnp.float32).astype(jnp.int32)
    comp = (j * P + P) - cnt

    comp_ref[0, pl.ds(off, P)] = comp

    carry_ref[0] = off + sel[0, P - 1]
    carry_ref[1] = tcnt + peq[0, P - 1]


def _compact_indices(e4):
    return pl.pallas_call(
        _k2_body,
        grid=(B, NB),
        in_specs=[pl.BlockSpec((1, NB, 1, P), lambda b, j: (b, 0, 0, 0))],
        out_specs=pl.BlockSpec((1, KP, 1), lambda b, j: (b, 0, 0)),
        out_shape=jax.ShapeDtypeStruct((B, KP, 1), jnp.int32),
        scratch_shapes=[
            pltpu.SMEM((4,), jnp.int32),
            pltpu.VMEM((P, P), jnp.float32),
            pltpu.VMEM((P, P), jnp.int32),
        ],
    )(e4)


def _sc_gather_rows(x_flat2, gidx):
    @pl.kernel(
        out_type=jax.ShapeDtypeStruct((BK, TC), jnp.float32),
        mesh=_vmesh(),
    )
    def k(x_hbm, i_hbm, o_hbm):
        def body(i_vmem, o_vmem):
            pltpu.sync_copy(x_hbm.at[i_vmem.at[0]], o_vmem)

        pltpu.emit_pipeline(
            body,
            grid=(BK // SC_WIN,),
            in_specs=[pl.BlockSpec((1, SC_WIN), lambda i: (0, i))],
            out_specs=[pl.BlockSpec((SC_WIN, TC), lambda i: (i, 0))],
            core_axis_name=("c", "s"),
            dimension_semantics=(pltpu.PARALLEL,),
        )(i_hbm, o_hbm)

    return k(x_flat2, gidx)


def kernel(x):
    xv = x.reshape(B, TC, N)
    x_flat, e4 = _energy_and_transpose(xv)
    comp = _compact_indices(e4)

    top_indices = comp[:, :K, 0]
    gidx = (top_indices + jnp.arange(B, dtype=jnp.int32)[:, None] * N)
    gathered = _sc_gather_rows(x_flat.reshape(BN, TC), gidx.reshape(1, BK))
    x_sparse = gathered.reshape(B, K, T, C)
    return x_sparse, top_indices


# speedup vs baseline: 15.9776x; 1.0040x over previous
import jax
import jax.numpy as jnp
from jax import lax
from jax.experimental import pallas as pl
from jax.experimental.pallas import tpu as pltpu
from jax.experimental.pallas import tpu_sc as plsc

B = 2
T = 8
C = 32
TC = T * C
H = W = 224
N = H * W
K = N // 2
P = 512
NB = N // P
BN = B * N
BK = B * K
KP = K + P
SC_WIN = 128

def _vmesh():
    return plsc.VectorSubcoreMesh(core_axis_name="c", subcore_axis_name="s")


P1 = 7168
NB1 = N // P1
RP = P1 // P


def _k1_body(x_ref, xf_ref, e_ref):
    xb = x_ref[0]
    xf_ref[0] = xb.T
    sq = xb * xb
    r3 = sq.reshape(T, C, P1)
    s = jnp.sum(r3, axis=1)
    rt = jnp.sqrt(s)
    e = jnp.sum(rt, axis=0, keepdims=True) * (1.0 / T)
    for r in range(RP):
        e_ref[0, r] = e[:, r * P:(r + 1) * P]


def _energy_and_transpose(xv):
    return pl.pallas_call(
        _k1_body,
        grid=(B, NB1),
        in_specs=[pl.BlockSpec((1, TC, P1), lambda b, j: (b, 0, j))],
        out_specs=[
            pl.BlockSpec((1, P1, TC), lambda b, j: (b, j, 0)),
            pl.BlockSpec((1, RP, 1, P), lambda b, j: (b, j, 0, 0)),
        ],
        out_shape=[
            jax.ShapeDtypeStruct((B, N, TC), jnp.float32),
            jax.ShapeDtypeStruct((B, NB, 1, P), jnp.float32),
        ],
    )(xv)


def _k2_body(e_ref, comp_ref, carry_ref, tri_ref, io_ref):
    j = pl.program_id(1)

    @pl.when(j == 0)
    def _():
        bits_all = lax.bitcast_convert_type(e_ref[0, :, 0, :], jnp.int32)

        def step(_, carry):
            lo, hi = carry
            mid = lo + (hi - lo) // 2
            g = jnp.sum((bits_all > mid).astype(jnp.int32))
            lo2 = jnp.where(g < K, lo, mid + 1)
            hi2 = jnp.where(g < K, mid, hi)
            return lo2, hi2

        thr, _ = lax.fori_loop(0, 31, step, (jnp.int32(0), jnp.int32(0x7F800000)))
        cnt_gt = jnp.sum((bits_all > thr).astype(jnp.int32))
        carry_ref[0] = 0
        carry_ref[1] = 0
        carry_ref[2] = thr
        carry_ref[3] = K - cnt_gt

        io = lax.broadcasted_iota(jnp.int32, (P, P), 0)
        jo = lax.broadcasted_iota(jnp.int32, (P, P), 1)
        io_ref[...] = io
        tri_ref[...] = jnp.where(io <= jo, 1.0, 0.0)

    off = carry_ref[0]
    tcnt = carry_ref[1]
    thr = carry_ref[2]
    r = carry_ref[3]

    bits = lax.bitcast_convert_type(e_ref[0, pl.ds(j, 1), 0, :], jnp.int32)
    m_gt = bits > thr
    m_eq = bits == thr

    both = jnp.concatenate(
        [m_gt.astype(jnp.float32), m_eq.astype(jnp.float32)], axis=0)
    pref = jnp.dot(both, tri_ref[...],
                   preferred_element_type=jnp.float32).astype(jnp.int32)
    pgt = pref[0:1]
    peq = pref[1:2]
    rr = jnp.maximum(r - tcnt, 0)
    sel = pgt + jnp.minimum(peq, rr)

    gmat = (jnp.broadcast_to(sel, (P, P)) > io_ref[...]).astype(jnp.float32)
    cnt = jnp.dot(gmat, jnp.ones((P, 1), jnp.float32),
                  preferred_element_type=jnp.float32).astype(jnp.int32)
    comp = (j * P + P) - cnt

    comp_ref[0, pl.ds(off, P)] = comp

    carry_ref[0] = off + sel[0, P - 1]
    carry_ref[1] = tcnt + peq[0, P - 1]


def _compact_indices(e4):
    return pl.pallas_call(
        _k2_body,
        grid=(B, NB),
        in_specs=[pl.BlockSpec((1, NB, 1, P), lambda b, j: (b, 0, 0, 0))],
        out_specs=pl.BlockSpec((1, KP, 1), lambda b, j: (b, 0, 0)),
        out_shape=jax.ShapeDtypeStruct((B, KP, 1), jnp.int32),
        scratch_shapes=[
            pltpu.SMEM((4,), jnp.int32),
            pltpu.VMEM((P, P), jnp.float32),
            pltpu.VMEM((P, P), jnp.int32),
        ],
    )(e4)


def _sc_gather_rows(x_flat2, gidx):
    @pl.kernel(
        out_type=jax.ShapeDtypeStruct((BK, TC), jnp.float32),
        mesh=_vmesh(),
    )
    def k(x_hbm, i_hbm, o_hbm):
        def body(i_vmem, o_vmem):
            pltpu.sync_copy(x_hbm.at[i_vmem.at[0]], o_vmem)

        pltpu.emit_pipeline(
            body,
            grid=(BK // SC_WIN,),
            in_specs=[pl.BlockSpec((1, SC_WIN), lambda i: (0, i))],
            out_specs=[pl.BlockSpec((SC_WIN, TC), lambda i: (i, 0))],
            core_axis_name=("c", "s"),
            dimension_semantics=(pltpu.PARALLEL,),
        )(i_hbm, o_hbm)

    return k(x_flat2, gidx)


def kernel(x):
    xv = x.reshape(B, TC, N)
    x_flat, e4 = _energy_and_transpose(xv)
    comp = _compact_indices(e4)

    top_indices = comp[:, :K, 0]
    gidx = (top_indices + jnp.arange(B, dtype=jnp.int32)[:, None] * N)
    gathered = _sc_gather_rows(x_flat.reshape(BN, TC), gidx.reshape(1, BK))
    x_sparse = gathered.reshape(B, K, T, C)
    return x_sparse, top_indices


# speedup vs baseline: 16.2318x; 1.0159x over previous
import jax
import jax.numpy as jnp
from jax import lax
from jax.experimental import pallas as pl
from jax.experimental.pallas import tpu as pltpu
from jax.experimental.pallas import tpu_sc as plsc

B = 2
T = 8
C = 32
TC = T * C
H = W = 224
N = H * W
K = N // 2
P = 512
NB = N // P
BN = B * N
BK = B * K
KP = K + P
SC_WIN = 128

def _vmesh():
    return plsc.VectorSubcoreMesh(core_axis_name="c", subcore_axis_name="s")


P1 = 7168
NB1 = N // P1
RP = P1 // P


def _k1_body(x_ref, xf_ref, e_ref):
    xb = x_ref[0]
    xf_ref[0] = xb.T
    sq = xb * xb
    r3 = sq.reshape(T, C, P1)
    s = jnp.sum(r3, axis=1)
    rt = jnp.sqrt(s)
    e = jnp.sum(rt, axis=0, keepdims=True) * (1.0 / T)
    for r in range(RP):
        e_ref[0, r] = e[:, r * P:(r + 1) * P]


def _energy_and_transpose(xv):
    return pl.pallas_call(
        _k1_body,
        grid=(B, NB1),
        in_specs=[pl.BlockSpec((1, TC, P1), lambda b, j: (b, 0, j))],
        out_specs=[
            pl.BlockSpec((1, P1, TC), lambda b, j: (b, j, 0)),
            pl.BlockSpec((1, RP, 1, P), lambda b, j: (b, j, 0, 0)),
        ],
        out_shape=[
            jax.ShapeDtypeStruct((B, N, TC), jnp.float32),
            jax.ShapeDtypeStruct((B, NB, 1, P), jnp.float32),
        ],
    )(xv)


def _k2_body(e_ref, comp_ref, carry_ref, tri_ref, io_ref, acc_ref):
    j = pl.program_id(1)

    @pl.when(j == 0)
    def _():
        bits_all = lax.bitcast_convert_type(e_ref[0, :, 0, :], jnp.int32)

        def step(_, carry):
            lo, hi = carry
            mid = lo + (hi - lo) // 2
            g = jnp.sum((bits_all > mid).astype(jnp.int32))
            lo2 = jnp.where(g < K, lo, mid + 1)
            hi2 = jnp.where(g < K, mid, hi)
            return lo2, hi2

        thr, _ = lax.fori_loop(0, 31, step, (jnp.int32(0), jnp.int32(0x7F800000)))
        cnt_gt = jnp.sum((bits_all > thr).astype(jnp.int32))
        carry_ref[0] = 0
        carry_ref[1] = 0
        carry_ref[2] = thr
        carry_ref[3] = K - cnt_gt

        io = lax.broadcasted_iota(jnp.int32, (P, P), 0)
        jo = lax.broadcasted_iota(jnp.int32, (P, P), 1)
        io_ref[...] = io
        tri_ref[...] = jnp.where(io <= jo, 1.0, 0.0)

    off = carry_ref[0]
    tcnt = carry_ref[1]
    thr = carry_ref[2]
    r = carry_ref[3]

    bits = lax.bitcast_convert_type(e_ref[0, pl.ds(j, 1), 0, :], jnp.int32)
    m_gt = bits > thr
    m_eq = bits == thr

    both = jnp.concatenate(
        [m_gt.astype(jnp.float32), m_eq.astype(jnp.float32)], axis=0)
    pref = jnp.dot(both, tri_ref[...],
                   preferred_element_type=jnp.float32).astype(jnp.int32)
    pgt = pref[0:1]
    peq = pref[1:2]
    rr = jnp.maximum(r - tcnt, 0)
    sel = pgt + jnp.minimum(peq, rr)

    gmat = (jnp.broadcast_to(sel, (P, P)) > io_ref[...]).astype(jnp.float32)
    cnt = jnp.dot(gmat, jnp.ones((P, 1), jnp.float32),
                  preferred_element_type=jnp.float32).astype(jnp.int32)
    comp = (j * P + P) - cnt

    acc_ref[pl.ds(off, P)] = comp

    carry_ref[0] = off + sel[0, P - 1]
    carry_ref[1] = tcnt + peq[0, P - 1]

    @pl.when(j == NB - 1)
    def _():
        for rr2 in range(KP // P):
            chunk = acc_ref[pl.ds(rr2 * P, P), :]
            comp_ref[0, :, rr2 * P:(rr2 + 1) * P] = chunk.T


def _compact_indices(e4):
    return pl.pallas_call(
        _k2_body,
        grid=(B, NB),
        in_specs=[pl.BlockSpec((1, NB, 1, P), lambda b, j: (b, 0, 0, 0))],
        out_specs=pl.BlockSpec((1, 1, KP), lambda b, j: (b, 0, 0)),
        out_shape=jax.ShapeDtypeStruct((B, 1, KP), jnp.int32),
        scratch_shapes=[
            pltpu.SMEM((4,), jnp.int32),
            pltpu.VMEM((P, P), jnp.float32),
            pltpu.VMEM((P, P), jnp.int32),
            pltpu.VMEM((KP, 1), jnp.int32),
        ],
    )(e4)


def _sc_gather_rows(x_flat2, gidx):
    @pl.kernel(
        out_type=jax.ShapeDtypeStruct((BK, TC), jnp.float32),
        mesh=_vmesh(),
    )
    def k(x_hbm, i_hbm, o_hbm):
        def body(i_vmem, o_vmem):
            pltpu.sync_copy(x_hbm.at[i_vmem.at[0]], o_vmem)

        pltpu.emit_pipeline(
            body,
            grid=(BK // SC_WIN,),
            in_specs=[pl.BlockSpec((1, SC_WIN), lambda i: (0, i))],
            out_specs=[pl.BlockSpec((SC_WIN, TC), lambda i: (i, 0))],
            core_axis_name=("c", "s"),
            dimension_semantics=(pltpu.PARALLEL,),
        )(i_hbm, o_hbm)

    return k(x_flat2, gidx)


def kernel(x):
    xv = x.reshape(B, TC, N)
    x_flat, e4 = _energy_and_transpose(xv)
    comp = _compact_indices(e4)

    top_indices = comp[:, 0, :K]
    gidx = (top_indices + jnp.arange(B, dtype=jnp.int32)[:, None] * N)
    gathered = _sc_gather_rows(x_flat.reshape(BN, TC), gidx.reshape(1, BK))
    x_sparse = gathered.reshape(B, K, T, C)
    return x_sparse, top_indices


# speedup vs baseline: 21.1661x; 1.3040x over previous
import jax
import jax.numpy as jnp
from jax import lax
from jax.experimental import pallas as pl
from jax.experimental.pallas import tpu as pltpu
from jax.experimental.pallas import tpu_sc as plsc

B = 2
T = 8
C = 32
TC = T * C
H = W = 224
N = H * W
K = N // 2
P = 512
NB = N // P
BN = B * N
BK = B * K
KP = K + P
SC_WIN = 128

def _vmesh():
    return plsc.VectorSubcoreMesh(core_axis_name="c", subcore_axis_name="s")


P1 = 7168
NB1 = N // P1
RP = P1 // P


def _k1_body(x_ref, xf_ref, e_ref):
    xb = x_ref[0]
    xf_ref[0] = xb.T
    sq = xb * xb
    r3 = sq.reshape(T, C, P1)
    s = jnp.sum(r3, axis=1)
    rt = jnp.sqrt(s)
    e = jnp.sum(rt, axis=0, keepdims=True) * (1.0 / T)
    for r in range(RP):
        e_ref[0, r] = e[:, r * P:(r + 1) * P]


def _energy_and_transpose(xv):
    return pl.pallas_call(
        _k1_body,
        grid=(B, NB1),
        in_specs=[pl.BlockSpec((1, TC, P1), lambda b, j: (b, 0, j))],
        out_specs=[
            pl.BlockSpec((1, P1, TC), lambda b, j: (b, j, 0)),
            pl.BlockSpec((1, RP, 1, P), lambda b, j: (b, j, 0, 0)),
        ],
        out_shape=[
            jax.ShapeDtypeStruct((B, N, TC), jnp.float32),
            jax.ShapeDtypeStruct((B, NB, 1, P), jnp.float32),
        ],
    )(xv)


def _k2_body(e_ref, comp_ref, carry_ref, tri_ref, io_ref, acc_ref):
    j = pl.program_id(1)

    @pl.when(j == 0)
    def _():
        bits_all = lax.bitcast_convert_type(e_ref[0, :, 0, :], jnp.int32)

        def step(_, carry):
            lo, hi = carry
            mid = lo + (hi - lo) // 2
            g = jnp.sum((bits_all > mid).astype(jnp.int32))
            lo2 = jnp.where(g < K, lo, mid + 1)
            hi2 = jnp.where(g < K, mid, hi)
            return lo2, hi2

        thr, _ = lax.fori_loop(0, 31, step, (jnp.int32(0), jnp.int32(0x7F800000)))
        cnt_gt = jnp.sum((bits_all > thr).astype(jnp.int32))
        carry_ref[0] = 0
        carry_ref[1] = 0
        carry_ref[2] = thr
        carry_ref[3] = K - cnt_gt

        io = lax.broadcasted_iota(jnp.int32, (P, P), 0)
        jo = lax.broadcasted_iota(jnp.int32, (P, P), 1)
        io_ref[...] = io
        tri_ref[...] = jnp.where(io <= jo, 1.0, 0.0)

    off = carry_ref[0]
    tcnt = carry_ref[1]
    thr = carry_ref[2]
    r = carry_ref[3]

    bits = lax.bitcast_convert_type(e_ref[0, pl.ds(j, 1), 0, :], jnp.int32)
    m_gt = bits > thr
    m_eq = bits == thr

    both = jnp.concatenate(
        [m_gt.astype(jnp.float32), m_eq.astype(jnp.float32)], axis=0)
    pref = jnp.dot(both, tri_ref[...],
                   preferred_element_type=jnp.float32).astype(jnp.int32)
    pgt = pref[0:1]
    peq = pref[1:2]
    rr = jnp.maximum(r - tcnt, 0)
    sel = pgt + jnp.minimum(peq, rr)

    gmat = (jnp.broadcast_to(sel, (P, P)) > io_ref[...]).astype(jnp.float32)
    cnt = jnp.dot(gmat, jnp.ones((P, 1), jnp.float32),
                  preferred_element_type=jnp.float32).astype(jnp.int32)
    comp = (j * P + P) - cnt

    acc_ref[pl.ds(off, P)] = comp

    carry_ref[0] = off + sel[0, P - 1]
    carry_ref[1] = tcnt + peq[0, P - 1]

    @pl.when(j == NB - 1)
    def _():
        for rr2 in range(KP // P):
            chunk = acc_ref[pl.ds(rr2 * P, P), :]
            comp_ref[0, :, rr2 * P:(rr2 + 1) * P] = chunk.T


def _compact_indices(e4b):
    return pl.pallas_call(
        _k2_body,
        grid=(1, NB),
        in_specs=[pl.BlockSpec((1, NB, 1, P), lambda b, j: (0, 0, 0, 0))],
        out_specs=pl.BlockSpec((1, 1, KP), lambda b, j: (0, 0, 0)),
        out_shape=jax.ShapeDtypeStruct((1, 1, KP), jnp.int32),
        scratch_shapes=[
            pltpu.SMEM((4,), jnp.int32),
            pltpu.VMEM((P, P), jnp.float32),
            pltpu.VMEM((P, P), jnp.int32),
            pltpu.VMEM((KP, 1), jnp.int32),
        ],
    )(e4b)


def _sc_gather_rows(x_flat2, gidx):
    @pl.kernel(
        out_type=jax.ShapeDtypeStruct((K, TC), jnp.float32),
        mesh=_vmesh(),
    )
    def k(x_hbm, i_hbm, o_hbm):
        def body(i_vmem, o_vmem):
            pltpu.sync_copy(x_hbm.at[i_vmem.at[0]], o_vmem)

        pltpu.emit_pipeline(
            body,
            grid=(K // SC_WIN,),
            in_specs=[pl.BlockSpec((1, SC_WIN), lambda i: (0, i))],
            out_specs=[pl.BlockSpec((SC_WIN, TC), lambda i: (i, 0))],
            core_axis_name=("c", "s"),
            dimension_semantics=(pltpu.PARALLEL,),
        )(i_hbm, o_hbm)

    return k(x_flat2, gidx)


def kernel(x):
    xv = x.reshape(B, TC, N)
    x_flat, e4 = _energy_and_transpose(xv)
    x_flat2 = x_flat.reshape(BN, TC)

    tops = []
    parts = []
    for b in range(B):
        comp_b = _compact_indices(e4[b:b + 1])
        top_b = comp_b[:, 0, :K]
        gidx_b = top_b + jnp.int32(b * N)
        parts.append(_sc_gather_rows(x_flat2, gidx_b))
        tops.append(top_b)

    top_indices = jnp.concatenate(tops, axis=0)
    x_sparse = jnp.stack(parts, axis=0).reshape(B, K, T, C)
    return x_sparse, top_indices
